# Initial kernel scaffold; baseline (speedup 1.0000x reference)
import jax
import jax.numpy as jnp
from jax.experimental import pallas as pl


def kernel(t, table, W, b):
    raise NotImplementedError("write your pallas kernel here")


# speedup vs baseline: 1.5837x; 1.5837x over previous
import functools
import math

import jax
import jax.numpy as jnp
from jax import lax
from jax.experimental import pallas as pl
from jax.experimental.pallas import tpu as pltpu
from jax.experimental.pallas import tpu_sc as plsc

N_HID = 62
MAX_LEN = 3000
N_ROWS = 4096
D_PAD = 128

_W_HMS = (3600.0 / 3661.0, 60.0 / 3661.0, 1.0 / 3661.0)

_NC = 2
_NS = 16
_NW = _NC * _NS
_RPW = N_ROWS // _NW


def _tc_table_body(table_ref, w_ref, b_ref, out_ref):
    prod = lax.dot_general(
        table_ref[...], w_ref[...],
        (((1,), (1,)), ((), ())),
        preferred_element_type=jnp.float32,
    )
    h = prod + b_ref[...]
    hp = jnp.concatenate(
        [h, jnp.zeros((MAX_LEN, D_PAD - N_HID), jnp.float32)], axis=1)
    out_ref[0] = hp * _W_HMS[0]
    out_ref[1] = hp * _W_HMS[1]
    out_ref[2] = hp * _W_HMS[2]


_tc_table = pl.pallas_call(
    _tc_table_body,
    out_shape=jax.ShapeDtypeStruct((3, MAX_LEN, D_PAD), jnp.float32),
)


def _sc_body(t_hbm, table3_hbm, out_hbm, tv, idxv, rows, acc, sem):
    wid = lax.axis_index("s") * _NC + lax.axis_index("c")
    base = wid * _RPW

    pltpu.sync_copy(t_hbm.at[pl.ds(base * 3, 3 * _RPW)], tv)

    lane = lax.iota(jnp.int32, 16)
    for g in range(3):
        for cc in range(8):
            c = g * 8 + cc
            off = ((lane + c * 16) % 3) * MAX_LEN
            idxv[g, pl.ds(cc * 16, 16)] = tv[pl.ds(c * 16, 16)] + off

    cps = [
        pltpu.async_copy(table3_hbm.at[idxv.at[g]],
                         rows.at[pl.ds(g * 128, 128)], sem)
        for g in range(3)
    ]
    for cp in cps:
        cp.wait()

    def body(i, carry):
        for c in range(4):
            s = pl.ds(c * 16, 16)
            acc[i, s] = rows[3 * i, s] + rows[3 * i + 1, s] + rows[3 * i + 2, s]
        return carry

    lax.fori_loop(0, _RPW, body, 0)

    pltpu.sync_copy(acc, out_hbm.at[pl.ds(base, _RPW)])


@functools.cache
def _sc_gather():
    return pl.kernel(
        _sc_body,
        out_type=jax.ShapeDtypeStruct((N_ROWS, D_PAD), jnp.float32),
        mesh=plsc.VectorSubcoreMesh(core_axis_name="c", subcore_axis_name="s"),
        scratch_types=[
            pltpu.VMEM((3 * _RPW,), jnp.int32),
            pltpu.VMEM((3, _RPW), jnp.int32),
            pltpu.VMEM((3 * _RPW, D_PAD), jnp.float32),
            pltpu.VMEM((_RPW, D_PAD), jnp.float32),
            pltpu.SemaphoreType.DMA,
        ],
    )


def kernel(t, table, W, b):
    table3 = _tc_table(table, W, b.reshape(1, N_HID))
    out = _sc_gather()(t.reshape(-1), table3.reshape(3 * MAX_LEN, D_PAD))
    return out[:, :N_HID]
